# baseline (speedup 1.0000x reference)
import jax
import jax.numpy as jnp
from jax import lax
from jax.experimental import pallas as pl
from jax.experimental.pallas import tpu as pltpu
from jax.experimental.pallas import tpu_sc as plsc

N = 10000
E = 320000
F1 = 128
F2 = 16
NP = 10080
NC, NS = 2, 16
NT = NC * NS
EPT = E // NT
K = 80
NBLK = EPT // K
RPS = NP // NS
ZK = 70
WACC = 32


def _sc_mesh():
    return plsc.VectorSubcoreMesh(core_axis_name="c", subcore_axis_name="s",
                                  num_cores=NC, num_subcores=NS)


def _make_sc_agg(feat, nphase, n_h, denom_phase, with_ee):
    cpf = feat // 16
    nout = nphase + (1 if denom_phase else 0)
    NPAIR = (NBLK - 1) // 2

    def body(*refs):
        (h_all_hbm, eler_hbm, epk_hbm, out_hbm,
         pk_v, eler_v, rows0_v, rows1_v, stg0_v, stg1_v, ee_all_v,
         sblk0_v, sblk1_v, dblk0_v, dblk1_v,
         gsem0, gsem1, ssem0, ssem1, acc_sh) = refs
        c = lax.axis_index("c")
        s = lax.axis_index("s")

        zf16 = jnp.zeros((16,), jnp.float32)
        zi16 = jnp.zeros((16,), jnp.int32)
        oi16 = jnp.ones((16,), jnp.int32)
        iota16 = lax.iota(jnp.int32, 16)
        rows_b = (rows0_v, rows1_v)
        stg_b = (stg0_v, stg1_v)
        sblk_b = (sblk0_v, sblk1_v)
        dblk_b = (dblk0_v, dblk1_v)
        gsem_b = (gsem0, gsem1)
        ssem_b = (ssem0, ssem1)

        def zero_my_slice():
            @pl.loop(0, ZK)
            def _(r):
                for q in range(WACC // 16):
                    stg0_v[r, pl.ds(16 * q, 16)] = zf16

            @pl.loop(0, RPS // ZK)
            def _(z):
                pltpu.sync_copy(stg0_v.at[pl.ds(0, ZK)],
                                acc_sh.at[pl.ds(s * RPS + z * ZK, ZK)])

        zero_my_slice()

        ebase = (c * NS + s) * EPT
        pltpu.sync_copy(epk_hbm.at[pl.ds(ebase, EPT)], pk_v)
        pltpu.sync_copy(eler_hbm, eler_v)

        plsc.subcore_barrier()

        def finish_phase(p, last):
            plsc.subcore_barrier()
            pltpu.sync_copy(acc_sh.at[pl.ds(s * RPS, RPS)],
                            out_hbm.at[p, c, pl.ds(s * RPS, RPS)])
            if not last:
                zero_my_slice()
                plsc.subcore_barrier()

        def load_sidx(off, b, p):
            for j in range(K // 16):
                s16 = jnp.bitwise_and(pk_v[pl.ds(off + 16 * j, 16)], 0xFFFF)
                if nphase > 1:
                    s16 = s16 * nphase + p
                sblk_b[b][pl.ds(16 * j, 16)] = s16

        def load_didx(off, b, shift_dst):
            del shift_dst
            for j in range(K // 16):
                d16 = lax.shift_right_logical(pk_v[pl.ds(off + 16 * j, 16)], 16)
                dblk_b[b][pl.ds(16 * j, 16)] = d16

        def start_gather(h_hbm, b):
            pltpu.async_copy(h_hbm.at[sblk_b[b]], rows_b[b], gsem_b[b])

        def wait_gather(h_hbm, b):
            pltpu.make_async_copy(h_hbm.at[sblk_b[b]], rows_b[b],
                                  gsem_b[b]).wait()

        def start_scatter(b):
            pltpu.async_copy(stg_b[b], acc_sh.at[dblk_b[b]], ssem_b[b],
                             add=True)

        def wait_scatter(b):
            pltpu.make_async_copy(stg_b[b], acc_sh.at[dblk_b[b]],
                                  ssem_b[b]).wait()

        def compute_ee(off, b):
            for j in range(K // 16):
                p16 = pk_v[pl.ds(off + 16 * j, 16)]
                src16 = jnp.bitwise_and(p16, 0xFFFF)
                dst16 = lax.shift_right_logical(p16, 16)
                el = plsc.load_gather(eler_v, [src16, zi16])
                er = plsc.load_gather(eler_v, [dst16, oi16])
                ssum = el + er
                ee_all_v[pl.ds(off + 16 * j, 16)] = jnp.exp(
                    jnp.maximum(ssum, 0.2 * ssum))

        def scale_rows(off, b, first):
            if first:
                compute_ee(off, b)

            @plsc.parallel_loop(0, K // 16, unroll=4)
            def _(g):
                ee16 = ee_all_v[pl.ds(off + 16 * g, 16)]
                for j in range(16):
                    r = 16 * g + j
                    a = ee16[j]
                    for q in range(cpf):
                        stg_b[b][r, pl.ds(16 * q, 16)] = (
                            rows_b[b][r, pl.ds(16 * q, 16)] * a)
                    if with_ee:
                        stg_b[b][r, pl.ds(feat, 16)] = jnp.where(
                            iota16 == 0, a, 0.0)

        def denom_rows(off, b, first):
            if first:
                compute_ee(off, b)

            @plsc.parallel_loop(0, K // 16, unroll=2)
            def _(g):
                ee16 = ee_all_v[pl.ds(off + 16 * g, 16)]
                for j in range(16):
                    r = 16 * g + j
                    a = ee16[j]
                    stg_b[b][r, pl.ds(0, 16)] = jnp.where(iota16 == 0, a, 0.0)
                    stg_b[b][r, pl.ds(16, 16)] = zf16

        def run_phase(h_hbm, p, shift_dst, process, first):
            if h_hbm is not None:
                load_sidx(0, 0, p)
                start_gather(h_hbm, 0)

            @pl.loop(0, NPAIR)
            def _(i):
                off0 = 2 * i * K
                if h_hbm is not None:
                    load_sidx(off0 + K, 1, p)
                    start_gather(h_hbm, 1)
                    wait_gather(h_hbm, 0)

                @pl.when(i > 0)
                def _():
                    wait_scatter(0)
                process(off0, 0, first)
                load_didx(off0, 0, shift_dst)
                start_scatter(0)
                if h_hbm is not None:
                    load_sidx(off0 + 2 * K, 0, p)
                    start_gather(h_hbm, 0)
                    wait_gather(h_hbm, 1)

                @pl.when(i > 0)
                def _():
                    wait_scatter(1)
                process(off0 + K, 1, first)
                load_didx(off0 + K, 1, shift_dst)
                start_scatter(1)

            toff = (NBLK - 1) * K
            if h_hbm is not None:
                wait_gather(h_hbm, 0)
            wait_scatter(0)
            process(toff, 0, first)
            load_didx(toff, 0, shift_dst)
            wait_scatter(1)
            pltpu.sync_copy(stg0_v, acc_sh.at[dblk0_v], add=True)

        for p in range(nphase):
            run_phase(h_all_hbm, p, False, scale_rows, first=(p == 0))
            finish_phase(p, last=(not denom_phase) and p == nphase - 1)

        if denom_phase:
            run_phase(None, 0, False, denom_rows, first=False)
            finish_phase(nphase, last=True)

    return pl.kernel(
        body,
        out_type=jax.ShapeDtypeStruct((nout, NC, NP, WACC), jnp.float32),
        mesh=_sc_mesh(),
        compiler_params=pltpu.CompilerParams(needs_layout_passes=False,
                                             use_tc_tiling_on_sc=False),
        scratch_types=[
            pltpu.VMEM((EPT,), jnp.int32),
            pltpu.VMEM((n_h, 2), jnp.float32),
            pltpu.VMEM((K, feat), jnp.float32),
            pltpu.VMEM((K, feat), jnp.float32),
            pltpu.VMEM((K, WACC), jnp.float32),
            pltpu.VMEM((K, WACC), jnp.float32),
            pltpu.VMEM((EPT,), jnp.float32),
            pltpu.VMEM((K,), jnp.int32),
            pltpu.VMEM((K,), jnp.int32),
            pltpu.VMEM((K,), jnp.int32),
            pltpu.VMEM((K,), jnp.int32),
            pltpu.SemaphoreType.DMA,
            pltpu.SemaphoreType.DMA,
            pltpu.SemaphoreType.DMA,
            pltpu.SemaphoreType.DMA,
            pltpu.VMEM_SHARED((NP, WACC), jnp.float32),
        ],
    )


def _tc1_body(x_ref, w_ref, a_ref, es_ref, ed_ref, h_ref, eler_ref, epk_ref):
    h = jnp.dot(x_ref[...], w_ref[...], preferred_element_type=jnp.float32)
    h_ref[...] = h
    eler_ref[...] = jnp.dot(h, a_ref[...], preferred_element_type=jnp.float32)
    epk_ref[...] = jnp.bitwise_or(jnp.left_shift(ed_ref[...], 16), es_ref[...])


def _tc2_body(p_ref, d_ref, w_ref, a_ref, h2_ref, eler_ref):
    parts = []
    for g in range(4):
        parts.append(p_ref[g, 0] + p_ref[g, 1])
    num = jnp.concatenate(parts, axis=1)
    den = (d_ref[0, 0] + d_ref[0, 1])[:, 0:1]
    h1 = jnp.maximum(num / (den + 1e-9), 0.0)
    h2 = jnp.dot(h1, w_ref[...], preferred_element_type=jnp.float32)
    h2_ref[...] = h2
    eler_ref[...] = jnp.dot(h2, a_ref[...], preferred_element_type=jnp.float32)


def _tc3_body(p_ref, o_ref):
    acc = p_ref[0, 0] + p_ref[0, 1]
    num = acc[:N, :F2]
    den = acc[:N, F2:F2 + 1]
    o_ref[...] = num / (den + 1e-9)


def kernel(x, edge_index, W1, al1, ar1, W2, al2, ar2):
    A1 = jnp.stack([al1, ar1], axis=1)
    A2 = jnp.stack([al2, ar2], axis=1)
    EP = 327680
    esrc = jnp.pad(edge_index[0], (0, EP - E))
    edst = jnp.pad(edge_index[1], (0, EP - E))
    EB = EP // (N // 1000)
    q = F1 // 4

    BR1 = 1000
    h1, eler1, epk = pl.pallas_call(
        _tc1_body,
        grid=(N // BR1,),
        in_specs=[
            pl.BlockSpec((BR1, F1), lambda i: (i, 0)),
            pl.BlockSpec((F1, F1), lambda i: (0, 0)),
            pl.BlockSpec((F1, 2), lambda i: (0, 0)),
            pl.BlockSpec((EB,), lambda i: (i,)),
            pl.BlockSpec((EB,), lambda i: (i,)),
        ],
        out_specs=[
            pl.BlockSpec((BR1, F1), lambda i: (i, 0)),
            pl.BlockSpec((BR1, 2), lambda i: (i, 0)),
            pl.BlockSpec((EB,), lambda i: (i,)),
        ],
        out_shape=[
            jax.ShapeDtypeStruct((N, F1), jnp.float32),
            jax.ShapeDtypeStruct((N, 2), jnp.float32),
            jax.ShapeDtypeStruct((EP,), jnp.int32),
        ],
    )(x, W1, A1, esrc, edst)

    h1v = h1.reshape(4 * N, q)
    part1 = _make_sc_agg(q, 4, N, True, False)(h1v, eler1, epk)

    BR2 = 1008
    h2, eler2 = pl.pallas_call(
        _tc2_body,
        grid=(NP // BR2,),
        in_specs=[
            pl.BlockSpec((4, 2, BR2, WACC), lambda i: (0, 0, i, 0)),
            pl.BlockSpec((1, 2, BR2, WACC), lambda i: (4, 0, i, 0)),
            pl.BlockSpec((F1, F2), lambda i: (0, 0)),
            pl.BlockSpec((F2, 2), lambda i: (0, 0)),
        ],
        out_specs=[
            pl.BlockSpec((BR2, F2), lambda i: (i, 0)),
            pl.BlockSpec((BR2, 2), lambda i: (i, 0)),
        ],
        out_shape=[
            jax.ShapeDtypeStruct((NP, F2), jnp.float32),
            jax.ShapeDtypeStruct((NP, 2), jnp.float32),
        ],
    )(part1, part1, W2, A2)

    part2 = _make_sc_agg(F2, 1, NP, False, True)(h2, eler2, epk)

    out = pl.pallas_call(
        _tc3_body,
        grid=(1,),
        in_specs=[pl.BlockSpec((1, 2, NP, WACC), lambda i: (0, 0, 0, 0))],
        out_specs=pl.BlockSpec((N, F2), lambda i: (0, 0)),
        out_shape=jax.ShapeDtypeStruct((N, F2), jnp.float32),
    )(part2)

    return out


# speedup vs baseline: 1.0024x; 1.0024x over previous
import jax
import jax.numpy as jnp
from jax import lax
from jax.experimental import pallas as pl
from jax.experimental.pallas import tpu as pltpu
from jax.experimental.pallas import tpu_sc as plsc

N = 10000
E = 320000
F1 = 128
F2 = 16
NP = 10080
NC, NS = 2, 16
NT = NC * NS
EPT = E // NT
K = 80
NBLK = EPT // K
RPS = NP // NS
ZK = 70
WACC = 32


def _sc_mesh():
    return plsc.VectorSubcoreMesh(core_axis_name="c", subcore_axis_name="s",
                                  num_cores=NC, num_subcores=NS)


def _make_sc_agg(feat, nphase, n_h, denom_phase, with_ee):
    cpf = feat // 16
    nout = nphase + (1 if denom_phase else 0)
    NPAIR = (NBLK - 1) // 2

    def body(*refs):
        (h_all_hbm, eler_hbm, epk_hbm, out_hbm,
         pk_v, eler_v, rows0_v, rows1_v, stg0_v, stg1_v, ee_all_v,
         sblk0_v, sblk1_v, dblk0_v, dblk1_v,
         gsem0, gsem1, ssem0, ssem1, acc_sh) = refs
        c = lax.axis_index("c")
        s = lax.axis_index("s")

        zf16 = jnp.zeros((16,), jnp.float32)
        zi16 = jnp.zeros((16,), jnp.int32)
        oi16 = jnp.ones((16,), jnp.int32)
        iota16 = lax.iota(jnp.int32, 16)
        rows_b = (rows0_v, rows1_v)
        stg_b = (stg0_v, stg1_v)
        sblk_b = (sblk0_v, sblk1_v)
        dblk_b = (dblk0_v, dblk1_v)
        gsem_b = (gsem0, gsem1)
        ssem_b = (ssem0, ssem1)

        def zero_my_slice():
            @pl.loop(0, ZK)
            def _(r):
                for q in range(WACC // 16):
                    stg0_v[r, pl.ds(16 * q, 16)] = zf16

            @pl.loop(0, RPS // ZK)
            def _(z):
                pltpu.sync_copy(stg0_v.at[pl.ds(0, ZK)],
                                acc_sh.at[pl.ds(s * RPS + z * ZK, ZK)])

        zero_my_slice()

        ebase = (c * NS + s) * EPT
        pltpu.sync_copy(epk_hbm.at[pl.ds(ebase, EPT)], pk_v)
        pltpu.sync_copy(eler_hbm, eler_v)

        plsc.subcore_barrier()

        def finish_phase(p, last):
            plsc.subcore_barrier()
            pltpu.sync_copy(acc_sh.at[pl.ds(s * RPS, RPS)],
                            out_hbm.at[p, c, pl.ds(s * RPS, RPS)])
            if not last:
                zero_my_slice()
                plsc.subcore_barrier()

        def load_sidx(off, b, p):
            for j in range(K // 16):
                s16 = jnp.bitwise_and(pk_v[pl.ds(off + 16 * j, 16)], 0xFFFF)
                if nphase > 1:
                    s16 = s16 * nphase + p
                sblk_b[b][pl.ds(16 * j, 16)] = s16

        def load_didx(off, b, shift_dst):
            del shift_dst
            for j in range(K // 16):
                d16 = lax.shift_right_logical(pk_v[pl.ds(off + 16 * j, 16)], 16)
                dblk_b[b][pl.ds(16 * j, 16)] = d16

        def start_gather(h_hbm, b):
            pltpu.async_copy(h_hbm.at[sblk_b[b]], rows_b[b], gsem_b[b])

        def wait_gather(h_hbm, b):
            pltpu.make_async_copy(h_hbm.at[sblk_b[b]], rows_b[b],
                                  gsem_b[b]).wait()

        def start_scatter(b):
            pltpu.async_copy(stg_b[b], acc_sh.at[dblk_b[b]], ssem_b[b],
                             add=True)

        def wait_scatter(b):
            pltpu.make_async_copy(stg_b[b], acc_sh.at[dblk_b[b]],
                                  ssem_b[b]).wait()

        def compute_ee(off, b):
            for j in range(K // 16):
                p16 = pk_v[pl.ds(off + 16 * j, 16)]
                src16 = jnp.bitwise_and(p16, 0xFFFF)
                dst16 = lax.shift_right_logical(p16, 16)
                el = plsc.load_gather(eler_v, [src16, zi16])
                er = plsc.load_gather(eler_v, [dst16, oi16])
                ssum = el + er
                ee_all_v[pl.ds(off + 16 * j, 16)] = jnp.exp(
                    jnp.maximum(ssum, 0.2 * ssum))

        def scale_rows(off, b, first):
            if first:
                compute_ee(off, b)

            @plsc.parallel_loop(0, K // 16, unroll=5)
            def _(g):
                ee16 = ee_all_v[pl.ds(off + 16 * g, 16)]
                for j in range(16):
                    r = 16 * g + j
                    a = ee16[j]
                    for q in range(cpf):
                        stg_b[b][r, pl.ds(16 * q, 16)] = (
                            rows_b[b][r, pl.ds(16 * q, 16)] * a)
                    if with_ee:
                        stg_b[b][r, pl.ds(feat, 16)] = jnp.where(
                            iota16 == 0, a, 0.0)

        def denom_rows(off, b, first):
            if first:
                compute_ee(off, b)

            @plsc.parallel_loop(0, K // 16, unroll=2)
            def _(g):
                ee16 = ee_all_v[pl.ds(off + 16 * g, 16)]
                for j in range(16):
                    r = 16 * g + j
                    a = ee16[j]
                    stg_b[b][r, pl.ds(0, 16)] = jnp.where(iota16 == 0, a, 0.0)
                    stg_b[b][r, pl.ds(16, 16)] = zf16

        def run_phase(h_hbm, p, shift_dst, process, first):
            if h_hbm is not None:
                load_sidx(0, 0, p)
                start_gather(h_hbm, 0)

            @pl.loop(0, NPAIR)
            def _(i):
                off0 = 2 * i * K
                if h_hbm is not None:
                    load_sidx(off0 + K, 1, p)
                    start_gather(h_hbm, 1)
                    wait_gather(h_hbm, 0)

                @pl.when(i > 0)
                def _():
                    wait_scatter(0)
                process(off0, 0, first)
                load_didx(off0, 0, shift_dst)
                start_scatter(0)
                if h_hbm is not None:
                    load_sidx(off0 + 2 * K, 0, p)
                    start_gather(h_hbm, 0)
                    wait_gather(h_hbm, 1)

                @pl.when(i > 0)
                def _():
                    wait_scatter(1)
                process(off0 + K, 1, first)
                load_didx(off0 + K, 1, shift_dst)
                start_scatter(1)

            toff = (NBLK - 1) * K
            if h_hbm is not None:
                wait_gather(h_hbm, 0)
            wait_scatter(0)
            process(toff, 0, first)
            load_didx(toff, 0, shift_dst)
            wait_scatter(1)
            pltpu.sync_copy(stg0_v, acc_sh.at[dblk0_v], add=True)

        for p in range(nphase):
            run_phase(h_all_hbm, p, False, scale_rows, first=(p == 0))
            finish_phase(p, last=(not denom_phase) and p == nphase - 1)

        if denom_phase:
            run_phase(None, 0, False, denom_rows, first=False)
            finish_phase(nphase, last=True)

    return pl.kernel(
        body,
        out_type=jax.ShapeDtypeStruct((nout, NC, NP, WACC), jnp.float32),
        mesh=_sc_mesh(),
        compiler_params=pltpu.CompilerParams(needs_layout_passes=False,
                                             use_tc_tiling_on_sc=False),
        scratch_types=[
            pltpu.VMEM((EPT,), jnp.int32),
            pltpu.VMEM((n_h, 2), jnp.float32),
            pltpu.VMEM((K, feat), jnp.float32),
            pltpu.VMEM((K, feat), jnp.float32),
            pltpu.VMEM((K, WACC), jnp.float32),
            pltpu.VMEM((K, WACC), jnp.float32),
            pltpu.VMEM((EPT,), jnp.float32),
            pltpu.VMEM((K,), jnp.int32),
            pltpu.VMEM((K,), jnp.int32),
            pltpu.VMEM((K,), jnp.int32),
            pltpu.VMEM((K,), jnp.int32),
            pltpu.SemaphoreType.DMA,
            pltpu.SemaphoreType.DMA,
            pltpu.SemaphoreType.DMA,
            pltpu.SemaphoreType.DMA,
            pltpu.VMEM_SHARED((NP, WACC), jnp.float32),
        ],
    )


def _tc1_body(x_ref, w_ref, a_ref, es_ref, ed_ref, h_ref, eler_ref, epk_ref):
    h = jnp.dot(x_ref[...], w_ref[...], preferred_element_type=jnp.float32)
    h_ref[...] = h
    eler_ref[...] = jnp.dot(h, a_ref[...], preferred_element_type=jnp.float32)
    epk_ref[...] = jnp.bitwise_or(jnp.left_shift(ed_ref[...], 16), es_ref[...])


def _tc2_body(p_ref, d_ref, w_ref, a_ref, h2_ref, eler_ref):
    parts = []
    for g in range(4):
        parts.append(p_ref[g, 0] + p_ref[g, 1])
    num = jnp.concatenate(parts, axis=1)
    den = (d_ref[0, 0] + d_ref[0, 1])[:, 0:1]
    h1 = jnp.maximum(num / (den + 1e-9), 0.0)
    h2 = jnp.dot(h1, w_ref[...], preferred_element_type=jnp.float32)
    h2_ref[...] = h2
    eler_ref[...] = jnp.dot(h2, a_ref[...], preferred_element_type=jnp.float32)


def _tc3_body(p_ref, o_ref):
    acc = p_ref[0, 0] + p_ref[0, 1]
    num = acc[:N, :F2]
    den = acc[:N, F2:F2 + 1]
    o_ref[...] = num / (den + 1e-9)


def kernel(x, edge_index, W1, al1, ar1, W2, al2, ar2):
    A1 = jnp.stack([al1, ar1], axis=1)
    A2 = jnp.stack([al2, ar2], axis=1)
    EP = 327680
    esrc = jnp.pad(edge_index[0], (0, EP - E))
    edst = jnp.pad(edge_index[1], (0, EP - E))
    EB = EP // (N // 1000)
    q = F1 // 4

    BR1 = 1000
    h1, eler1, epk = pl.pallas_call(
        _tc1_body,
        grid=(N // BR1,),
        in_specs=[
            pl.BlockSpec((BR1, F1), lambda i: (i, 0)),
            pl.BlockSpec((F1, F1), lambda i: (0, 0)),
            pl.BlockSpec((F1, 2), lambda i: (0, 0)),
            pl.BlockSpec((EB,), lambda i: (i,)),
            pl.BlockSpec((EB,), lambda i: (i,)),
        ],
        out_specs=[
            pl.BlockSpec((BR1, F1), lambda i: (i, 0)),
            pl.BlockSpec((BR1, 2), lambda i: (i, 0)),
            pl.BlockSpec((EB,), lambda i: (i,)),
        ],
        out_shape=[
            jax.ShapeDtypeStruct((N, F1), jnp.float32),
            jax.ShapeDtypeStruct((N, 2), jnp.float32),
            jax.ShapeDtypeStruct((EP,), jnp.int32),
        ],
    )(x, W1, A1, esrc, edst)

    h1v = h1.reshape(4 * N, q)
    part1 = _make_sc_agg(q, 4, N, True, False)(h1v, eler1, epk)

    BR2 = 1008
    h2, eler2 = pl.pallas_call(
        _tc2_body,
        grid=(NP // BR2,),
        in_specs=[
            pl.BlockSpec((4, 2, BR2, WACC), lambda i: (0, 0, i, 0)),
            pl.BlockSpec((1, 2, BR2, WACC), lambda i: (4, 0, i, 0)),
            pl.BlockSpec((F1, F2), lambda i: (0, 0)),
            pl.BlockSpec((F2, 2), lambda i: (0, 0)),
        ],
        out_specs=[
            pl.BlockSpec((BR2, F2), lambda i: (i, 0)),
            pl.BlockSpec((BR2, 2), lambda i: (i, 0)),
        ],
        out_shape=[
            jax.ShapeDtypeStruct((NP, F2), jnp.float32),
            jax.ShapeDtypeStruct((NP, 2), jnp.float32),
        ],
    )(part1, part1, W2, A2)

    part2 = _make_sc_agg(F2, 1, NP, False, True)(h2, eler2, epk)

    out = pl.pallas_call(
        _tc3_body,
        grid=(1,),
        in_specs=[pl.BlockSpec((1, 2, NP, WACC), lambda i: (0, 0, 0, 0))],
        out_specs=pl.BlockSpec((N, F2), lambda i: (0, 0)),
        out_shape=jax.ShapeDtypeStruct((N, F2), jnp.float32),
    )(part2)

    return out
